# baseline (speedup 1.0000x reference)
import jax
import jax.numpy as jnp
from jax.experimental import pallas as pl
from jax.experimental.pallas import tpu as pltpu

N = 10000
K = 16
IN_FEATS = 256
OUT_FEATS = 256

BN = 200
NBUF = 8
S = N // BN
BNK = BN * K
NBULK = (S - 1) * BN


def _fused_kernel(x_hbm, wt_ref, b_ref, o_hbm, xbuf, obuf, sems, osems):
    i = pl.program_id(0)

    def issue(c):
        b = jax.lax.rem(c, NBUF)
        pltpu.make_async_copy(
            x_hbm.at[pl.ds(c * BNK, BNK), :],
            xbuf.at[b],
            sems.at[b],
        ).start()

    @pl.when(i == 0)
    def _prologue():
        for c in range(min(NBUF, S)):
            issue(c)

    b = jax.lax.rem(i, NBUF)
    pltpu.make_async_copy(
        x_hbm.at[pl.ds(i * BNK, BNK), :], xbuf.at[b], sems.at[b]
    ).wait()

    x = xbuf[b]
    y = jnp.dot(x, wt_ref[...], preferred_element_type=jnp.float32)
    m = jnp.max(y.reshape(BN, K, OUT_FEATS), axis=1)
    obuf[pl.ds(i * BN, BN), :] = m + b_ref[...]

    @pl.when(i + NBUF < S)
    def _refill():
        issue(i + NBUF)

    bulk = pltpu.make_async_copy(
        obuf.at[pl.ds(0, NBULK), :], o_hbm.at[pl.ds(0, NBULK), :], osems.at[0]
    )
    tail = pltpu.make_async_copy(
        obuf.at[pl.ds(NBULK, BN), :], o_hbm.at[pl.ds(NBULK, BN), :], osems.at[1]
    )

    @pl.when(i == S - 2)
    def _write_bulk():
        bulk.start()

    @pl.when(i == S - 1)
    def _write_tail():
        tail.start()
        bulk.wait()
        tail.wait()


@jax.jit
def kernel(neighbour, W, b):
    wt = W.T
    b2 = b.reshape(1, OUT_FEATS)
    x2 = neighbour.reshape(N * K, IN_FEATS)
    return pl.pallas_call(
        _fused_kernel,
        grid=(S,),
        in_specs=[
            pl.BlockSpec(memory_space=pl.ANY),
            pl.BlockSpec((IN_FEATS, OUT_FEATS), lambda i: (0, 0)),
            pl.BlockSpec((1, OUT_FEATS), lambda i: (0, 0)),
        ],
        out_specs=pl.BlockSpec(memory_space=pl.ANY),
        out_shape=jax.ShapeDtypeStruct((N, OUT_FEATS), jnp.float32),
        scratch_shapes=[
            pltpu.VMEM((NBUF, BNK, IN_FEATS), jnp.float32),
            pltpu.VMEM((N, OUT_FEATS), jnp.float32),
            pltpu.SemaphoreType.DMA((NBUF,)),
            pltpu.SemaphoreType.DMA((2,)),
        ],
    )(x2, wt, b2)


# speedup vs baseline: 1.0012x; 1.0012x over previous
import jax
import jax.numpy as jnp
from jax.experimental import pallas as pl
from jax.experimental.pallas import tpu as pltpu

N = 10000
K = 16
IN_FEATS = 256
OUT_FEATS = 256

BN = 200
NBUF = 8
S = N // BN
BNK = BN * K
NBULK = (S - 1) * BN


def _fused_kernel(x_hbm, wt_ref, b_ref, o_hbm, xbuf, obuf, sems, osems):
    i = pl.program_id(0)

    def issue(c):
        b = jax.lax.rem(c, NBUF)
        pltpu.make_async_copy(
            x_hbm.at[pl.ds(c * BNK, BNK), :],
            xbuf.at[b],
            sems.at[b],
        ).start()

    @pl.when(i == 0)
    def _prologue():
        for c in range(min(NBUF, S)):
            issue(c)

    b = jax.lax.rem(i, NBUF)
    pltpu.make_async_copy(
        x_hbm.at[pl.ds(i * BNK, BNK), :], xbuf.at[b], sems.at[b]
    ).wait()

    x = xbuf[b]
    y = jnp.dot(x, wt_ref[...], preferred_element_type=jnp.float32)
    m = jnp.max(y.reshape(BN, K, OUT_FEATS), axis=1)
    obuf[i] = m + b_ref[...]

    @pl.when(i + NBUF < S)
    def _refill():
        issue(i + NBUF)

    bulk = pltpu.make_async_copy(
        obuf.at[pl.ds(0, S - 1)], o_hbm.at[pl.ds(0, S - 1)], osems.at[0]
    )
    tail = pltpu.make_async_copy(
        obuf.at[pl.ds(S - 1, 1)], o_hbm.at[pl.ds(S - 1, 1)], osems.at[1]
    )

    @pl.when(i == S - 2)
    def _write_bulk():
        bulk.start()

    @pl.when(i == S - 1)
    def _write_tail():
        tail.start()
        bulk.wait()
        tail.wait()


@jax.jit
def kernel(neighbour, W, b):
    wt = W.T
    b2 = b.reshape(1, OUT_FEATS)
    x2 = neighbour.reshape(N * K, IN_FEATS)
    out = pl.pallas_call(
        _fused_kernel,
        grid=(S,),
        in_specs=[
            pl.BlockSpec(memory_space=pl.ANY),
            pl.BlockSpec((IN_FEATS, OUT_FEATS), lambda i: (0, 0)),
            pl.BlockSpec((1, OUT_FEATS), lambda i: (0, 0)),
        ],
        out_specs=pl.BlockSpec(memory_space=pl.ANY),
        out_shape=jax.ShapeDtypeStruct((S, BN, OUT_FEATS), jnp.float32),
        scratch_shapes=[
            pltpu.VMEM((NBUF, BNK, IN_FEATS), jnp.float32),
            pltpu.VMEM((S, BN, OUT_FEATS), jnp.float32),
            pltpu.SemaphoreType.DMA((NBUF,)),
            pltpu.SemaphoreType.DMA((2,)),
        ],
    )(x2, wt, b2)
    return out.reshape(N, OUT_FEATS)
